# Initial kernel scaffold; baseline (speedup 1.0000x reference)
import functools

import jax
import jax.numpy as jnp
from jax import lax
from jax.experimental import pallas as pl
from jax.experimental.pallas import tpu as pltpu
from jax.experimental.pallas import tpu_sc as plsc

N = 10000
E = 320000
D = 128
FH = 128
OUTD = 21
OP = 32
NC = 2
NS = 16
NW = NC * NS
NPAD = 10240
ROWS_PER_TILE = N // NS
EPW = E // NW

_MESH = plsc.VectorSubcoreMesh(core_axis_name="c", subcore_axis_name="s")


def _rsqrt16(x):
    i = plsc.bitcast(x, jnp.int32)
    i = jnp.full((16,), 0x5F3759DF, jnp.int32) - (i >> 1)
    y = plsc.bitcast(i, jnp.float32)
    hx = 0.5 * x
    for _ in range(3):
        y = y * (1.5 - hx * y * y)
    return y


KD = 80
EPT_DEG = E // NS


@functools.partial(
    pl.kernel,
    out_type=jax.ShapeDtypeStruct((NPAD,), jnp.float32),
    mesh=_MESH,
    scratch_types=[
        pltpu.VMEM_SHARED((NPAD,), jnp.float32),
        pltpu.VMEM((KD,), jnp.int32),
        pltpu.VMEM((KD,), jnp.float32),
        pltpu.VMEM((NPAD // NS,), jnp.float32),
        pltpu.VMEM((NPAD // NW,), jnp.float32),
    ],
)
def _deg_dis(dst_hbm, dis_hbm, deg_sp, dv, ones_v, zv, wv):
    c = lax.axis_index("c")
    s = lax.axis_index("s")
    for j in range(NPAD // NS // 16):
        zv[pl.ds(j * 16, 16)] = jnp.zeros((16,), jnp.float32)
    for j in range(KD // 16):
        ones_v[pl.ds(j * 16, 16)] = jnp.ones((16,), jnp.float32)
    pltpu.sync_copy(zv, deg_sp.at[pl.ds(s * (NPAD // NS), NPAD // NS)])
    plsc.subcore_barrier()

    def body(i, carry):
        base = s * EPT_DEG + i * KD
        pltpu.sync_copy(dst_hbm.at[pl.ds(base, KD)], dv)
        pltpu.sync_copy(ones_v, deg_sp.at[dv], add=True)
        return carry

    lax.fori_loop(0, EPT_DEG // KD, body, 0)
    plsc.subcore_barrier()

    w = c * NS + s
    nb = NPAD // NW
    pltpu.sync_copy(deg_sp.at[pl.ds(w * nb, nb)], wv)
    for g in range(nb // 16):
        d = wv[pl.ds(g * 16, 16)] + 1.0
        wv[pl.ds(g * 16, 16)] = _rsqrt16(d)
    pltpu.sync_copy(wv, dis_hbm.at[pl.ds(w * nb, nb)])


def _make_agg(F, K):
    nchunk = EPW // K

    @functools.partial(
        pl.kernel,
        out_type=jax.ShapeDtypeStruct((NC, N, F), jnp.float32),
        mesh=_MESH,
        scratch_types=[
            pltpu.VMEM_SHARED((N, F), jnp.float32),
            pltpu.VMEM((K,), jnp.int32),
            pltpu.VMEM((K,), jnp.int32),
            pltpu.VMEM((K, F), jnp.float32),
            pltpu.SemaphoreType.DMA,
        ],
    )
    def _agg(h_hbm, src_hbm, dst_hbm, zero_hbm, out_hbm, acc_sp, sv, dv, rows, sem):
        c = lax.axis_index("c")
        s = lax.axis_index("s")
        pltpu.sync_copy(zero_hbm, acc_sp.at[pl.ds(s * ROWS_PER_TILE, ROWS_PER_TILE)])
        plsc.subcore_barrier()
        w = c * NS + s

        def body(i, carry):
            base = w * EPW + i * K
            pltpu.sync_copy(src_hbm.at[pl.ds(base, K)], sv)
            pltpu.sync_copy(dst_hbm.at[pl.ds(base, K)], dv)
            pltpu.async_copy(h_hbm.at[sv], rows, sem).wait()
            pltpu.sync_copy(rows, acc_sp.at[dv], add=True)
            return carry

        lax.fori_loop(0, nchunk, body, 0)
        plsc.subcore_barrier()
        pltpu.sync_copy(
            acc_sp.at[pl.ds(s * ROWS_PER_TILE, ROWS_PER_TILE)],
            out_hbm.at[c, pl.ds(s * ROWS_PER_TILE, ROWS_PER_TILE)],
        )

    return _agg


_agg128 = _make_agg(FH, 80)
_agg32 = _make_agg(OP, 80)


RB = 1000
GRID = N // RB


def _tc0_body(x_ref, w_ref, o_ref):
    o_ref[...] = jnp.dot(x_ref[...], w_ref[...], preferred_element_type=jnp.float32)


def _tc1_body(m_ref, d_ref, o_ref):
    o_ref[...] = m_ref[...] * d_ref[...]


def _tc2_body(p_ref, h_ref, d_ref, b_ref, w_ref, o_ref):
    d = d_ref[...]
    y = jnp.maximum((p_ref[0] + p_ref[1] + h_ref[...]) * d + b_ref[...], 0.0)
    o_ref[...] = jnp.dot(y, w_ref[...], preferred_element_type=jnp.float32) * d


def _tc3_body(p_ref, h_ref, d_ref, b_ref, o_ref):
    o_ref[...] = (p_ref[0] + p_ref[1] + h_ref[...]) * d_ref[...] + b_ref[...]


def _row_spec(F):
    return pl.BlockSpec((RB, F), lambda i: (i, 0))


def _full_spec(shape):
    n = len(shape)
    return pl.BlockSpec(shape, lambda i, _n=n: (0,) * _n)


def _parts_spec(F):
    return pl.BlockSpec((NC, RB, F), lambda i: (0, i, 0))


def _tc0(x, W1):
    return pl.pallas_call(
        _tc0_body,
        grid=(GRID,),
        in_specs=[_row_spec(D), _full_spec((D, FH))],
        out_specs=_row_spec(FH),
        out_shape=jax.ShapeDtypeStruct((N, FH), jnp.float32),
    )(x, W1)


def _tc1(m1, dis):
    return pl.pallas_call(
        _tc1_body,
        grid=(GRID,),
        in_specs=[_row_spec(FH), _row_spec(1)],
        out_specs=_row_spec(FH),
        out_shape=jax.ShapeDtypeStruct((N, FH), jnp.float32),
    )(m1, dis)


def _tc2(p1, h1p, dis, b1r, W2p):
    return pl.pallas_call(
        _tc2_body,
        grid=(GRID,),
        in_specs=[
            _parts_spec(FH),
            _row_spec(FH),
            _row_spec(1),
            _full_spec((1, FH)),
            _full_spec((FH, OP)),
        ],
        out_specs=_row_spec(OP),
        out_shape=jax.ShapeDtypeStruct((N, OP), jnp.float32),
    )(p1, h1p, dis, b1r, W2p)


def _tc3(p2, h2p, dis, b2r):
    return pl.pallas_call(
        _tc3_body,
        grid=(GRID,),
        in_specs=[
            _parts_spec(OP),
            _row_spec(OP),
            _row_spec(1),
            _full_spec((1, OP)),
        ],
        out_specs=_row_spec(OP),
        out_shape=jax.ShapeDtypeStruct((N, OP), jnp.float32),
    )(p2, h2p, dis, b2r)


def kernel(x, edge_index, W1, b1, W2, b2):
    x = x.astype(jnp.float32)
    src = edge_index[0].astype(jnp.int32)
    dst = edge_index[1].astype(jnp.int32)

    dis_pad = _deg_dis(dst)
    dis = dis_pad[:N].reshape(N, 1)
    m1 = _tc0(x, W1)
    h1p = _tc1(m1, dis)

    z128 = jnp.zeros((ROWS_PER_TILE, FH), jnp.float32)
    p1 = _agg128(h1p, src, dst, z128)

    W2p = jnp.zeros((FH, OP), jnp.float32).at[:, :OUTD].set(W2)
    b1r = b1.reshape(1, FH)
    h2p = _tc2(p1, h1p, dis, b1r, W2p)

    z32 = jnp.zeros((ROWS_PER_TILE, OP), jnp.float32)
    p2 = _agg32(h2p, src, dst, z32)

    b2r = jnp.zeros((1, OP), jnp.float32).at[0, :OUTD].set(b2)
    outp = _tc3(p2, h2p, dis, b2r)
    return outp[:, :OUTD]


# speedup vs baseline: 12.0923x; 12.0923x over previous
import functools

import jax
import jax.numpy as jnp
from jax import lax
from jax.experimental import pallas as pl
from jax.experimental.pallas import tpu as pltpu
from jax.experimental.pallas import tpu_sc as plsc

N = 10000
E = 320000
D = 128
FH = 128
OUTD = 21
OP = 32
NC = 2
NS = 16
NW = NC * NS
NPAD = 10240
ROWS_PER_TILE = NPAD // NS
EPW = E // NW

_MESH = plsc.VectorSubcoreMesh(
    core_axis_name="c", subcore_axis_name="s", num_cores=NC, num_subcores=NS
)


def _rsqrt16(x):
    y = 1.0 / x
    hx = 0.5 * x
    for _ in range(26):
        y = y * (1.5 - hx * y * y)
    return y


KD = 80
EPT_DEG = E // NS


@functools.partial(
    pl.kernel,
    out_type=jax.ShapeDtypeStruct((NPAD,), jnp.float32),
    mesh=_MESH,
    scratch_types=[
        pltpu.VMEM_SHARED((NPAD,), jnp.float32),
        pltpu.VMEM((KD,), jnp.int32),
        pltpu.VMEM((KD,), jnp.float32),
        pltpu.VMEM((NPAD // NS,), jnp.float32),
        pltpu.VMEM((NPAD // NW,), jnp.float32),
    ],
)
def _deg_dis(dst_hbm, dis_hbm, deg_sp, dv, ones_v, zv, wv):
    c = lax.axis_index("c")
    s = lax.axis_index("s")
    for j in range(NPAD // NS // 16):
        zv[pl.ds(j * 16, 16)] = jnp.zeros((16,), jnp.float32)
    for j in range(KD // 16):
        ones_v[pl.ds(j * 16, 16)] = jnp.ones((16,), jnp.float32)
    pltpu.sync_copy(zv, deg_sp.at[pl.ds(s * (NPAD // NS), NPAD // NS)])
    plsc.subcore_barrier()

    def body(i, carry):
        base = s * EPT_DEG + i * KD
        pltpu.sync_copy(dst_hbm.at[pl.ds(base, KD)], dv)
        pltpu.sync_copy(ones_v, deg_sp.at[dv], add=True)
        return carry

    lax.fori_loop(0, EPT_DEG // KD, body, 0)
    plsc.subcore_barrier()

    w = c * NS + s
    nb = NPAD // NW
    pltpu.sync_copy(deg_sp.at[pl.ds(w * nb, nb)], wv)
    for g in range(nb // 16):
        d = wv[pl.ds(g * 16, 16)] + 1.0
        wv[pl.ds(g * 16, 16)] = _rsqrt16(d)
    pltpu.sync_copy(wv, dis_hbm.at[pl.ds(w * nb, nb)])


def _make_agg(F, K):
    nchunk = EPW // K

    @functools.partial(
        pl.kernel,
        out_type=jax.ShapeDtypeStruct((NC, NPAD, F), jnp.float32),
        mesh=_MESH,
        scratch_types=[
            pltpu.VMEM_SHARED((NPAD, F), jnp.float32),
            pltpu.VMEM((K,), jnp.int32),
            pltpu.VMEM((K,), jnp.int32),
            pltpu.VMEM((K, F), jnp.float32),
            pltpu.SemaphoreType.DMA,
        ],
    )
    def _agg(h_hbm, src_hbm, dst_hbm, zero_hbm, out_hbm, acc_sp, sv, dv, rows, sem):
        c = lax.axis_index("c")
        s = lax.axis_index("s")
        pltpu.sync_copy(zero_hbm, acc_sp.at[pl.ds(s * ROWS_PER_TILE, ROWS_PER_TILE)])
        plsc.subcore_barrier()
        w = c * NS + s

        def body(i, carry):
            base = w * EPW + i * K
            pltpu.sync_copy(src_hbm.at[pl.ds(base, K)], sv)
            pltpu.sync_copy(dst_hbm.at[pl.ds(base, K)], dv)
            pltpu.async_copy(h_hbm.at[sv], rows, sem).wait()
            pltpu.sync_copy(rows, acc_sp.at[dv], add=True)
            return carry

        lax.fori_loop(0, nchunk, body, 0)
        plsc.subcore_barrier()
        pltpu.sync_copy(
            acc_sp.at[pl.ds(s * ROWS_PER_TILE, ROWS_PER_TILE)],
            out_hbm.at[c, pl.ds(s * ROWS_PER_TILE, ROWS_PER_TILE)],
        )

    return _agg


_agg128 = _make_agg(FH, 80)


RB = 1000
GRID = N // RB


def _tc0_body(x_ref, w_ref, o_ref):
    o_ref[...] = jnp.dot(x_ref[...], w_ref[...], preferred_element_type=jnp.float32)


def _tc1_body(m_ref, d_ref, o_ref):
    o_ref[...] = m_ref[...] * d_ref[...]


def _tc2_body(p_ref, h_ref, d_ref, b_ref, o_ref):
    d = d_ref[...]
    y = jnp.maximum((p_ref[0] + p_ref[1] + h_ref[...]) * d + b_ref[...], 0.0)
    o_ref[...] = y * d


def _tc3_body(p_ref, h_ref, d_ref, b_ref, w_ref, o_ref):
    agg = (p_ref[0] + p_ref[1] + h_ref[...]) * d_ref[...]
    o_ref[...] = (
        jnp.dot(agg, w_ref[...], preferred_element_type=jnp.float32) + b_ref[...]
    )


def _row_spec(F):
    return pl.BlockSpec((RB, F), lambda i: (i, 0))


def _full_spec(shape):
    n = len(shape)
    return pl.BlockSpec(shape, lambda i, _n=n: (0,) * _n)


def _parts_spec(F):
    return pl.BlockSpec((NC, RB, F), lambda i: (0, i, 0))


def _tc0(x, W1):
    return pl.pallas_call(
        _tc0_body,
        grid=(GRID,),
        in_specs=[_row_spec(D), _full_spec((D, FH))],
        out_specs=_row_spec(FH),
        out_shape=jax.ShapeDtypeStruct((N, FH), jnp.float32),
    )(x, W1)


def _tc1(m1, dis):
    return pl.pallas_call(
        _tc1_body,
        grid=(GRID,),
        in_specs=[_row_spec(FH), _row_spec(1)],
        out_specs=_row_spec(FH),
        out_shape=jax.ShapeDtypeStruct((N, FH), jnp.float32),
    )(m1, dis)


def _tc2(p1, h1p, dis, b1r):
    return pl.pallas_call(
        _tc2_body,
        grid=(GRID,),
        in_specs=[
            _parts_spec(FH),
            _row_spec(FH),
            _row_spec(1),
            _full_spec((1, FH)),
        ],
        out_specs=_row_spec(FH),
        out_shape=jax.ShapeDtypeStruct((N, FH), jnp.float32),
    )(p1, h1p, dis, b1r)


def _tc3(p2, y1p, dis, b2r, W2p):
    return pl.pallas_call(
        _tc3_body,
        grid=(GRID,),
        in_specs=[
            _parts_spec(FH),
            _row_spec(FH),
            _row_spec(1),
            _full_spec((1, OP)),
            _full_spec((FH, OP)),
        ],
        out_specs=_row_spec(OP),
        out_shape=jax.ShapeDtypeStruct((N, OP), jnp.float32),
    )(p2, y1p, dis, b2r, W2p)


def kernel(x, edge_index, W1, b1, W2, b2):
    x = x.astype(jnp.float32)
    src = edge_index[0].astype(jnp.int32)
    dst = edge_index[1].astype(jnp.int32)

    dis_pad = _deg_dis(dst)
    dis = dis_pad[:N].reshape(N, 1)
    m1 = _tc0(x, W1)
    h1p = _tc1(m1, dis)

    z128 = jnp.zeros((ROWS_PER_TILE, FH), jnp.float32)
    p1 = _agg128(h1p, src, dst, z128)

    b1r = b1.reshape(1, FH)
    y1p = _tc2(p1, h1p, dis, b1r)

    p2 = _agg128(y1p, src, dst, z128)

    W2p = jnp.zeros((FH, OP), jnp.float32).at[:, :OUTD].set(W2)
    b2r = jnp.zeros((1, OP), jnp.float32).at[0, :OUTD].set(b2)
    outp = _tc3(p2, y1p, dis, b2r, W2p)
    return outp[:, :OUTD]


# speedup vs baseline: 15.3277x; 1.2676x over previous
import functools

import jax
import jax.numpy as jnp
from jax import lax
from jax.experimental import pallas as pl
from jax.experimental.pallas import tpu as pltpu
from jax.experimental.pallas import tpu_sc as plsc

N = 10000
E = 320000
D = 128
FH = 128
OUTD = 21
OP = 32
NC = 2
NS = 16
NW = NC * NS
NPAD = 10240
ROWS_PER_TILE = NPAD // NS
EPW = E // NW

_MESH = plsc.VectorSubcoreMesh(
    core_axis_name="c", subcore_axis_name="s", num_cores=NC, num_subcores=NS
)


def _rsqrt16(x):
    y = 1.0 / x
    hx = 0.5 * x
    for _ in range(26):
        y = y * (1.5 - hx * y * y)
    return y


KD = 80
EPT_DEG = E // NS


@functools.partial(
    pl.kernel,
    out_type=jax.ShapeDtypeStruct((NPAD,), jnp.float32),
    mesh=_MESH,
    scratch_types=[
        pltpu.VMEM_SHARED((NPAD,), jnp.float32),
        pltpu.VMEM((KD,), jnp.int32),
        pltpu.VMEM((KD,), jnp.int32),
        pltpu.VMEM((KD,), jnp.float32),
        pltpu.VMEM((NPAD // NS,), jnp.float32),
        pltpu.VMEM((NPAD // NW,), jnp.float32),
        pltpu.SemaphoreType.DMA,
        pltpu.SemaphoreType.DMA,
        pltpu.SemaphoreType.DMA,
        pltpu.SemaphoreType.DMA,
    ],
)
def _deg_dis(dst_hbm, dis_hbm, deg_sp, dv0, dv1, ones_v, zv, wv, is0, is1, ss0, ss1):
    c = lax.axis_index("c")
    s = lax.axis_index("s")
    for j in range(NPAD // NS // 16):
        zv[pl.ds(j * 16, 16)] = jnp.zeros((16,), jnp.float32)
    for j in range(KD // 16):
        ones_v[pl.ds(j * 16, 16)] = jnp.ones((16,), jnp.float32)
    pltpu.sync_copy(zv, deg_sp.at[pl.ds(s * (NPAD // NS), NPAD // NS)])
    plsc.subcore_barrier()

    base = s * (EPT_DEG // KD)
    nchunk = EPT_DEG // KD
    last = base + nchunk

    def iload(k, dv):
        pltpu.sync_copy(dst_hbm.at[pl.ds(k * KD, KD)], dv)

    def body(p, carry):
        iload(base + p, dv0)
        pltpu.sync_copy(ones_v, deg_sp.at[dv0], add=True)
        return carry

    lax.fori_loop(0, nchunk, body, 0)
    plsc.subcore_barrier()

    w = c * NS + s
    nb = NPAD // NW
    pltpu.sync_copy(deg_sp.at[pl.ds(w * nb, nb)], wv)
    for g in range(nb // 16):
        d = wv[pl.ds(g * 16, 16)] + 1.0
        wv[pl.ds(g * 16, 16)] = _rsqrt16(d)
    pltpu.sync_copy(wv, dis_hbm.at[pl.ds(w * nb, nb)])


def _make_agg(F, K):
    tot_chunks = E // K
    lo = (tot_chunks // NW) & ~1
    hi = (tot_chunks - 16 * lo) // 16

    @functools.partial(
        pl.kernel,
        out_type=jax.ShapeDtypeStruct((NC, NPAD, F), jnp.float32),
        mesh=_MESH,
        scratch_types=[
            pltpu.VMEM_SHARED((NPAD, F), jnp.float32),
            pltpu.VMEM((K,), jnp.int32),
            pltpu.VMEM((K,), jnp.int32),
            pltpu.VMEM((K, F), jnp.float32),
            pltpu.VMEM((K,), jnp.int32),
            pltpu.VMEM((K,), jnp.int32),
            pltpu.VMEM((K, F), jnp.float32),
            pltpu.SemaphoreType.DMA,
            pltpu.SemaphoreType.DMA,
            pltpu.SemaphoreType.DMA,
            pltpu.SemaphoreType.DMA,
            pltpu.SemaphoreType.DMA,
            pltpu.SemaphoreType.DMA,
        ],
    )
    def _agg(h_hbm, src_hbm, dst_hbm, zero_hbm, out_hbm, acc_sp,
             sv0, dv0, rows0, sv1, dv1, rows1, is0, is1, gs0, gs1, ss0, ss1):
        c = lax.axis_index("c")
        s = lax.axis_index("s")
        pltpu.sync_copy(zero_hbm, acc_sp.at[pl.ds(s * ROWS_PER_TILE, ROWS_PER_TILE)])
        plsc.subcore_barrier()
        w = c * NS + s
        cnt = jnp.where(w < 16, lo, hi)
        base = jnp.where(w < 16, w * lo, 16 * lo + (w - 16) * hi)
        last = base + cnt

        def iload(k, sv, dv):
            pltpu.sync_copy(src_hbm.at[pl.ds(k * K, K)], sv)
            pltpu.sync_copy(dst_hbm.at[pl.ds(k * K, K)], dv)

        iload(base, sv0, dv0)
        pltpu.async_copy(h_hbm.at[sv0], rows0, gs0).wait()

        def body(p, carry):
            e = base + 2 * p
            iload(e + 1, sv1, dv1)
            d_g1 = pltpu.async_copy(h_hbm.at[sv1], rows1, gs1)
            pltpu.sync_copy(rows0, acc_sp.at[dv0], add=True)
            k0 = jnp.minimum(e + 2, last - 2)
            iload(k0, sv0, dv0)
            d_g1.wait()
            d_g0 = pltpu.async_copy(h_hbm.at[sv0], rows0, gs0)
            pltpu.sync_copy(rows1, acc_sp.at[dv1], add=True)
            d_g0.wait()
            return carry

        lax.fori_loop(0, cnt // 2, body, 0)
        plsc.subcore_barrier()
        pltpu.sync_copy(
            acc_sp.at[pl.ds(s * ROWS_PER_TILE, ROWS_PER_TILE)],
            out_hbm.at[c, pl.ds(s * ROWS_PER_TILE, ROWS_PER_TILE)],
        )

    return _agg


_agg128 = _make_agg(FH, 80)


RB = 1000
GRID = N // RB


def _tc0_body(x_ref, w_ref, o_ref):
    o_ref[...] = jnp.dot(x_ref[...], w_ref[...], preferred_element_type=jnp.float32)


def _tc1_body(m_ref, d_ref, o_ref):
    o_ref[...] = m_ref[...] * d_ref[...]


def _tc2_body(p_ref, h_ref, d_ref, b_ref, o_ref):
    d = d_ref[...]
    y = jnp.maximum((p_ref[0] + p_ref[1] + h_ref[...]) * d + b_ref[...], 0.0)
    o_ref[...] = y * d


def _tc3_body(p_ref, h_ref, d_ref, b_ref, w_ref, o_ref):
    agg = (p_ref[0] + p_ref[1] + h_ref[...]) * d_ref[...]
    o_ref[...] = (
        jnp.dot(agg, w_ref[...], preferred_element_type=jnp.float32) + b_ref[...]
    )


def _row_spec(F):
    return pl.BlockSpec((RB, F), lambda i: (i, 0))


def _full_spec(shape):
    n = len(shape)
    return pl.BlockSpec(shape, lambda i, _n=n: (0,) * _n)


def _parts_spec(F):
    return pl.BlockSpec((NC, RB, F), lambda i: (0, i, 0))


def _tc0(x, W1):
    return pl.pallas_call(
        _tc0_body,
        grid=(GRID,),
        in_specs=[_row_spec(D), _full_spec((D, FH))],
        out_specs=_row_spec(FH),
        out_shape=jax.ShapeDtypeStruct((N, FH), jnp.float32),
    )(x, W1)


def _tc1(m1, dis):
    return pl.pallas_call(
        _tc1_body,
        grid=(GRID,),
        in_specs=[_row_spec(FH), _row_spec(1)],
        out_specs=_row_spec(FH),
        out_shape=jax.ShapeDtypeStruct((N, FH), jnp.float32),
    )(m1, dis)


def _tc2(p1, h1p, dis, b1r):
    return pl.pallas_call(
        _tc2_body,
        grid=(GRID,),
        in_specs=[
            _parts_spec(FH),
            _row_spec(FH),
            _row_spec(1),
            _full_spec((1, FH)),
        ],
        out_specs=_row_spec(FH),
        out_shape=jax.ShapeDtypeStruct((N, FH), jnp.float32),
    )(p1, h1p, dis, b1r)


def _tc3(p2, y1p, dis, b2r, W2p):
    return pl.pallas_call(
        _tc3_body,
        grid=(GRID,),
        in_specs=[
            _parts_spec(FH),
            _row_spec(FH),
            _row_spec(1),
            _full_spec((1, OP)),
            _full_spec((FH, OP)),
        ],
        out_specs=_row_spec(OP),
        out_shape=jax.ShapeDtypeStruct((N, OP), jnp.float32),
    )(p2, y1p, dis, b2r, W2p)


def kernel(x, edge_index, W1, b1, W2, b2):
    x = x.astype(jnp.float32)
    src = edge_index[0].astype(jnp.int32)
    dst = edge_index[1].astype(jnp.int32)

    dis_pad = _deg_dis(dst)
    dis = dis_pad[:N].reshape(N, 1)
    m1 = _tc0(x, W1)
    h1p = _tc1(m1, dis)

    z128 = jnp.zeros((ROWS_PER_TILE, FH), jnp.float32)
    p1 = _agg128(h1p, src, dst, z128)

    b1r = b1.reshape(1, FH)
    y1p = _tc2(p1, h1p, dis, b1r)

    p2 = _agg128(y1p, src, dst, z128)

    W2p = jnp.zeros((FH, OP), jnp.float32).at[:, :OUTD].set(W2)
    b2r = jnp.zeros((1, OP), jnp.float32).at[0, :OUTD].set(b2)
    outp = _tc3(p2, y1p, dis, b2r, W2p)
    return outp[:, :OUTD]


# speedup vs baseline: 23.4300x; 1.5286x over previous
import functools

import jax
import jax.numpy as jnp
from jax import lax
from jax.experimental import pallas as pl
from jax.experimental.pallas import tpu as pltpu
from jax.experimental.pallas import tpu_sc as plsc

N = 10000
E = 320000
D = 128
FH = 128
OUTD = 21
OP = 32
NC = 2
NS = 16
NW = NC * NS
NPAD = 10240
ROWS_PER_TILE = NPAD // NS
EPW = E // NW

_MESH = plsc.VectorSubcoreMesh(
    core_axis_name="c", subcore_axis_name="s", num_cores=NC, num_subcores=NS
)


def _rsqrt16(x):
    y = 1.0 / x
    hx = 0.5 * x
    for _ in range(26):
        y = y * (1.5 - hx * y * y)
    return y


KD = 80
NCHUNK = E // KD
EPT_DEG = E // NS


@functools.partial(
    pl.kernel,
    out_type=jax.ShapeDtypeStruct((NPAD,), jnp.float32),
    mesh=_MESH,
    scratch_types=[
        pltpu.VMEM_SHARED((NPAD,), jnp.float32),
        pltpu.VMEM((256, KD), jnp.int32),
        pltpu.VMEM((KD,), jnp.float32),
        pltpu.VMEM((NPAD // NS,), jnp.float32),
        pltpu.VMEM((NPAD // NW,), jnp.float32),
    ],
)
def _deg_dis(dst2_hbm, dis_hbm, deg_sp, dv_all, ones_v, zv, wv):
    c = lax.axis_index("c")
    s = lax.axis_index("s")
    for j in range(NPAD // NS // 16):
        zv[pl.ds(j * 16, 16)] = jnp.zeros((16,), jnp.float32)
    for j in range(KD // 16):
        ones_v[pl.ds(j * 16, 16)] = jnp.ones((16,), jnp.float32)
    pltpu.sync_copy(zv, deg_sp.at[pl.ds(s * (NPAD // NS), NPAD // NS)])
    plsc.subcore_barrier()

    cnt = jnp.where(s < 12, 248, 256)
    base = jnp.where(s < 12, s * 248, 2976 + (s - 12) * 256)
    lbase = jnp.minimum(base, NCHUNK - 256)
    off = base - lbase
    pltpu.sync_copy(dst2_hbm.at[pl.ds(lbase, 256)], dv_all)

    def body(j, carry):
        pltpu.sync_copy(ones_v, deg_sp.at[dv_all.at[off + j]], add=True)
        return carry

    lax.fori_loop(0, cnt, body, 0)
    plsc.subcore_barrier()

    w = c * NS + s
    nb = NPAD // NW
    pltpu.sync_copy(deg_sp.at[pl.ds(w * nb, nb)], wv)
    for g in range(nb // 16):
        d = wv[pl.ds(g * 16, 16)] + 1.0
        wv[pl.ds(g * 16, 16)] = _rsqrt16(d)
    pltpu.sync_copy(wv, dis_hbm.at[pl.ds(w * nb, nb)])


def _make_agg(F, K):
    @functools.partial(
        pl.kernel,
        out_type=jax.ShapeDtypeStruct((NC, NPAD, F), jnp.float32),
        mesh=_MESH,
        scratch_types=[
            pltpu.VMEM_SHARED((NPAD, F), jnp.float32),
            pltpu.VMEM((128 * K,), jnp.int32),
            pltpu.VMEM((128, K), jnp.int32),
            pltpu.VMEM((K, F), jnp.float32),
            pltpu.VMEM((K, F), jnp.float32),
            pltpu.SemaphoreType.DMA,
            pltpu.SemaphoreType.DMA,
        ],
    )
    def _agg(h_hbm, src_hbm, dst2_hbm, zero_hbm, out_hbm, acc_sp,
             sv_all, dv_all, rows0, rows1, gs0, gs1):
        c = lax.axis_index("c")
        s = lax.axis_index("s")
        pltpu.sync_copy(zero_hbm, acc_sp.at[pl.ds(s * ROWS_PER_TILE, ROWS_PER_TILE)])
        w = c * NS + s
        cnt = jnp.where(w < 12, 120, 128)
        base = jnp.where(w < 12, w * 120, 1440 + (w - 12) * 128)
        pltpu.sync_copy(src_hbm.at[pl.ds(base * K, 128 * K)], sv_all)
        pltpu.sync_copy(dst2_hbm.at[pl.ds(base, 128)], dv_all)
        plsc.subcore_barrier()

        def gather(rel, rows, sem):
            return pltpu.async_copy(
                h_hbm.at[sv_all.at[pl.ds(rel * K, K)]], rows, sem
            )

        gather(0, rows0, gs0).wait()

        def body(p, carry):
            r0 = 2 * p
            d_g1 = gather(r0 + 1, rows1, gs1)
            pltpu.sync_copy(rows0, acc_sp.at[dv_all.at[r0]], add=True)
            d_g1.wait()
            r2 = jnp.minimum(r0 + 2, cnt - 2)
            d_g0 = gather(r2, rows0, gs0)
            pltpu.sync_copy(rows1, acc_sp.at[dv_all.at[r0 + 1]], add=True)
            d_g0.wait()
            return carry

        lax.fori_loop(0, cnt // 2, body, 0)
        plsc.subcore_barrier()
        pltpu.sync_copy(
            acc_sp.at[pl.ds(s * ROWS_PER_TILE, ROWS_PER_TILE)],
            out_hbm.at[c, pl.ds(s * ROWS_PER_TILE, ROWS_PER_TILE)],
        )

    return _agg


_agg128 = _make_agg(FH, 80)


RB = 1000
GRID = N // RB


def _tc0_body(x_ref, w_ref, o_ref):
    o_ref[...] = jnp.dot(x_ref[...], w_ref[...], preferred_element_type=jnp.float32)


def _tc1_body(m_ref, d_ref, o_ref):
    o_ref[...] = m_ref[...] * d_ref[...]


def _tc2_body(p_ref, h_ref, d_ref, b_ref, o_ref):
    d = d_ref[...]
    y = jnp.maximum((p_ref[0] + p_ref[1] + h_ref[...]) * d + b_ref[...], 0.0)
    o_ref[...] = y * d


def _tc3_body(p_ref, h_ref, d_ref, b_ref, w_ref, o_ref):
    agg = (p_ref[0] + p_ref[1] + h_ref[...]) * d_ref[...]
    o_ref[...] = (
        jnp.dot(agg, w_ref[...], preferred_element_type=jnp.float32) + b_ref[...]
    )


def _row_spec(F):
    return pl.BlockSpec((RB, F), lambda i: (i, 0))


def _full_spec(shape):
    n = len(shape)
    return pl.BlockSpec(shape, lambda i, _n=n: (0,) * _n)


def _parts_spec(F):
    return pl.BlockSpec((NC, RB, F), lambda i: (0, i, 0))


def _tc0(x, W1):
    return pl.pallas_call(
        _tc0_body,
        grid=(GRID,),
        in_specs=[_row_spec(D), _full_spec((D, FH))],
        out_specs=_row_spec(FH),
        out_shape=jax.ShapeDtypeStruct((N, FH), jnp.float32),
    )(x, W1)


def _tc1(m1, dis):
    return pl.pallas_call(
        _tc1_body,
        grid=(GRID,),
        in_specs=[_row_spec(FH), _row_spec(1)],
        out_specs=_row_spec(FH),
        out_shape=jax.ShapeDtypeStruct((N, FH), jnp.float32),
    )(m1, dis)


def _tc2(p1, h1p, dis, b1r):
    return pl.pallas_call(
        _tc2_body,
        grid=(GRID,),
        in_specs=[
            _parts_spec(FH),
            _row_spec(FH),
            _row_spec(1),
            _full_spec((1, FH)),
        ],
        out_specs=_row_spec(FH),
        out_shape=jax.ShapeDtypeStruct((N, FH), jnp.float32),
    )(p1, h1p, dis, b1r)


def _tc3(p2, y1p, dis, b2r, W2p):
    return pl.pallas_call(
        _tc3_body,
        grid=(GRID,),
        in_specs=[
            _parts_spec(FH),
            _row_spec(FH),
            _row_spec(1),
            _full_spec((1, OP)),
            _full_spec((FH, OP)),
        ],
        out_specs=_row_spec(OP),
        out_shape=jax.ShapeDtypeStruct((N, OP), jnp.float32),
    )(p2, y1p, dis, b2r, W2p)


def kernel(x, edge_index, W1, b1, W2, b2):
    x = x.astype(jnp.float32)
    src = edge_index[0].astype(jnp.int32)
    dst = edge_index[1].astype(jnp.int32)

    dst2 = dst.reshape(NCHUNK, KD)
    dis_pad = _deg_dis(dst2)
    dis = dis_pad[:N].reshape(N, 1)
    m1 = _tc0(x, W1)
    h1p = _tc1(m1, dis)

    z128 = jnp.zeros((ROWS_PER_TILE, FH), jnp.float32)
    p1 = _agg128(h1p, src, dst2, z128)

    b1r = b1.reshape(1, FH)
    y1p = _tc2(p1, h1p, dis, b1r)

    p2 = _agg128(y1p, src, dst2, z128)

    W2p = jnp.zeros((FH, OP), jnp.float32).at[:, :OUTD].set(W2)
    b2r = jnp.zeros((1, OP), jnp.float32).at[0, :OUTD].set(b2)
    outp = _tc3(p2, y1p, dis, b2r, W2p)
    return outp[:, :OUTD]


# speedup vs baseline: 26.4717x; 1.1298x over previous
import functools

import jax
import jax.numpy as jnp
from jax import lax
from jax.experimental import pallas as pl
from jax.experimental.pallas import tpu as pltpu
from jax.experimental.pallas import tpu_sc as plsc

N = 10000
E = 320000
D = 128
FH = 128
OUTD = 21
OP = 32
NC = 2
NS = 16
NW = NC * NS
NPAD = 10240
ROWS_PER_TILE = NPAD // NS
EPW = E // NW

_MESH = plsc.VectorSubcoreMesh(
    core_axis_name="c", subcore_axis_name="s", num_cores=NC, num_subcores=NS
)


def _rsqrt16(x):
    y = 1.0 / x
    hx = 0.5 * x
    for _ in range(26):
        y = y * (1.5 - hx * y * y)
    return y


KD = 80
NCHUNK = E // KD
EPT_DEG = E // NS


@functools.partial(
    pl.kernel,
    out_type=jax.ShapeDtypeStruct((NPAD,), jnp.float32),
    mesh=_MESH,
    scratch_types=[
        pltpu.VMEM_SHARED((NPAD,), jnp.float32),
        pltpu.VMEM((256, KD), jnp.int32),
        pltpu.VMEM((KD,), jnp.float32),
        pltpu.VMEM((NPAD // NS,), jnp.float32),
        pltpu.VMEM((NPAD // NW,), jnp.float32),
    ],
)
def _deg_dis(dst2_hbm, dis_hbm, deg_sp, dv_all, ones_v, zv, wv):
    c = lax.axis_index("c")
    s = lax.axis_index("s")
    for j in range(NPAD // NS // 16):
        zv[pl.ds(j * 16, 16)] = jnp.zeros((16,), jnp.float32)
    for j in range(KD // 16):
        ones_v[pl.ds(j * 16, 16)] = jnp.ones((16,), jnp.float32)
    pltpu.sync_copy(zv, deg_sp.at[pl.ds(s * (NPAD // NS), NPAD // NS)])
    plsc.subcore_barrier()

    cnt = jnp.where(s < 12, 248, 256)
    base = jnp.where(s < 12, s * 248, 2976 + (s - 12) * 256)
    lbase = jnp.minimum(base, NCHUNK - 256)
    off = base - lbase
    pltpu.sync_copy(dst2_hbm.at[pl.ds(lbase, 256)], dv_all)

    def body(j, carry):
        pltpu.sync_copy(ones_v, deg_sp.at[dv_all.at[off + j]], add=True)
        return carry

    lax.fori_loop(0, cnt, body, 0)
    plsc.subcore_barrier()

    w = c * NS + s
    nb = NPAD // NW
    pltpu.sync_copy(deg_sp.at[pl.ds(w * nb, nb)], wv)
    for g in range(nb // 16):
        d = wv[pl.ds(g * 16, 16)] + 1.0
        wv[pl.ds(g * 16, 16)] = _rsqrt16(d)
    pltpu.sync_copy(wv, dis_hbm.at[pl.ds(w * nb, nb)])


def _make_agg(F, K, tc_tiling=True):
    @functools.partial(
        pl.kernel,
        out_type=jax.ShapeDtypeStruct((NC, NPAD, F), jnp.float32),
        mesh=_MESH,
        compiler_params=pltpu.CompilerParams(use_tc_tiling_on_sc=tc_tiling),
        scratch_types=[
            pltpu.VMEM_SHARED((NPAD, F), jnp.float32),
            pltpu.VMEM((128 * K,), jnp.int32),
            pltpu.VMEM((128, K), jnp.int32),
            pltpu.VMEM((K, F), jnp.float32),
            pltpu.VMEM((K, F), jnp.float32),
            pltpu.SemaphoreType.DMA,
            pltpu.SemaphoreType.DMA,
        ],
    )
    def _agg(h_hbm, src_hbm, dst2_hbm, zero_hbm, out_hbm, acc_sp,
             sv_all, dv_all, rows0, rows1, gs0, gs1):
        c = lax.axis_index("c")
        s = lax.axis_index("s")
        pltpu.sync_copy(zero_hbm, acc_sp.at[pl.ds(s * ROWS_PER_TILE, ROWS_PER_TILE)])
        w = c * NS + s
        cnt = jnp.where(w < 12, 120, 128)
        base = jnp.where(w < 12, w * 120, 1440 + (w - 12) * 128)
        pltpu.sync_copy(src_hbm.at[pl.ds(base * K, 128 * K)], sv_all)
        pltpu.sync_copy(dst2_hbm.at[pl.ds(base, 128)], dv_all)
        plsc.subcore_barrier()

        def gather(rel, rows, sem):
            return pltpu.async_copy(
                h_hbm.at[sv_all.at[pl.ds(rel * K, K)]], rows, sem
            )

        gather(0, rows0, gs0).wait()

        def body(p, carry):
            r0 = 2 * p
            d_g1 = gather(r0 + 1, rows1, gs1)
            pltpu.sync_copy(rows0, acc_sp.at[dv_all.at[r0]], add=True)
            d_g1.wait()
            r2 = jnp.minimum(r0 + 2, cnt - 2)
            d_g0 = gather(r2, rows0, gs0)
            pltpu.sync_copy(rows1, acc_sp.at[dv_all.at[r0 + 1]], add=True)
            d_g0.wait()
            return carry

        lax.fori_loop(0, cnt // 2, body, 0)
        plsc.subcore_barrier()
        pltpu.sync_copy(
            acc_sp.at[pl.ds(s * ROWS_PER_TILE, ROWS_PER_TILE)],
            out_hbm.at[c, pl.ds(s * ROWS_PER_TILE, ROWS_PER_TILE)],
        )

    return _agg


_agg128 = _make_agg(FH, 80)
_agg32 = _make_agg(OP, 80, tc_tiling=False)


RB = 1000
GRID = N // RB


def _tc0_body(x_ref, w_ref, o_ref):
    o_ref[...] = jnp.dot(x_ref[...], w_ref[...], preferred_element_type=jnp.float32)


def _tc1_body(m_ref, d_ref, o_ref):
    o_ref[...] = m_ref[...] * d_ref[...]


def _tc2_body(p_ref, h_ref, d_ref, b_ref, w_ref, o_ref):
    d = d_ref[...]
    y = jnp.maximum((p_ref[0] + p_ref[1] + h_ref[...]) * d + b_ref[...], 0.0)
    o_ref[...] = jnp.dot(y, w_ref[...], preferred_element_type=jnp.float32) * d


def _tc3_body(p_ref, h_ref, d_ref, b_ref, o_ref):
    o_ref[...] = (p_ref[0] + p_ref[1] + h_ref[...]) * d_ref[...] + b_ref[...]


def _row_spec(F):
    return pl.BlockSpec((RB, F), lambda i: (i, 0))


def _full_spec(shape):
    n = len(shape)
    return pl.BlockSpec(shape, lambda i, _n=n: (0,) * _n)


def _parts_spec(F):
    return pl.BlockSpec((NC, RB, F), lambda i: (0, i, 0))


def _tc0(x, W1):
    return pl.pallas_call(
        _tc0_body,
        grid=(GRID,),
        in_specs=[_row_spec(D), _full_spec((D, FH))],
        out_specs=_row_spec(FH),
        out_shape=jax.ShapeDtypeStruct((N, FH), jnp.float32),
    )(x, W1)


def _tc1(m1, dis):
    return pl.pallas_call(
        _tc1_body,
        grid=(GRID,),
        in_specs=[_row_spec(FH), _row_spec(1)],
        out_specs=_row_spec(FH),
        out_shape=jax.ShapeDtypeStruct((N, FH), jnp.float32),
    )(m1, dis)


def _tc2(p1, h1p, dis, b1r, W2p):
    return pl.pallas_call(
        _tc2_body,
        grid=(GRID,),
        in_specs=[
            _parts_spec(FH),
            _row_spec(FH),
            _row_spec(1),
            _full_spec((1, FH)),
            _full_spec((FH, OP)),
        ],
        out_specs=_row_spec(OP),
        out_shape=jax.ShapeDtypeStruct((N, OP), jnp.float32),
    )(p1, h1p, dis, b1r, W2p)


def _tc3(p2, h2p, dis, b2r):
    return pl.pallas_call(
        _tc3_body,
        grid=(GRID,),
        in_specs=[
            _parts_spec(OP),
            _row_spec(OP),
            _row_spec(1),
            _full_spec((1, OP)),
        ],
        out_specs=_row_spec(OP),
        out_shape=jax.ShapeDtypeStruct((N, OP), jnp.float32),
    )(p2, h2p, dis, b2r)


def kernel(x, edge_index, W1, b1, W2, b2):
    x = x.astype(jnp.float32)
    src = edge_index[0].astype(jnp.int32)
    dst = edge_index[1].astype(jnp.int32)

    dst2 = dst.reshape(NCHUNK, KD)
    dis_pad = _deg_dis(dst2)
    dis = dis_pad[:N].reshape(N, 1)
    m1 = _tc0(x, W1)
    h1p = _tc1(m1, dis)

    z128 = jnp.zeros((ROWS_PER_TILE, FH), jnp.float32)
    p1 = _agg128(h1p, src, dst2, z128)

    b1r = b1.reshape(1, FH)
    W2p = jnp.zeros((FH, OP), jnp.float32).at[:, :OUTD].set(W2)
    h2p = _tc2(p1, h1p, dis, b1r, W2p)

    z32 = jnp.zeros((ROWS_PER_TILE, OP), jnp.float32)
    p2 = _agg32(h2p, src, dst2, z32)

    b2r = jnp.zeros((1, OP), jnp.float32).at[0, :OUTD].set(b2)
    outp = _tc3(p2, h2p, dis, b2r)
    return outp[:, :OUTD]


# speedup vs baseline: 28.4908x; 1.0763x over previous
import functools

import jax
import jax.numpy as jnp
from jax import lax
from jax.experimental import pallas as pl
from jax.experimental.pallas import tpu as pltpu
from jax.experimental.pallas import tpu_sc as plsc

N = 10000
E = 320000
D = 128
FH = 128
OUTD = 21
OP = 32
NC = 2
NS = 16
NW = NC * NS
NPAD = 10240
ROWS_PER_TILE = NPAD // NS
EPW = E // NW

_MESH = plsc.VectorSubcoreMesh(
    core_axis_name="c", subcore_axis_name="s", num_cores=NC, num_subcores=NS
)


def _rsqrt16(x):
    y = 1.0 / x
    hx = 0.5 * x
    for _ in range(26):
        y = y * (1.5 - hx * y * y)
    return y


KD = 80
NCHUNK = E // KD
EPT_DEG = E // NS


@functools.partial(
    pl.kernel,
    out_type=jax.ShapeDtypeStruct((NPAD,), jnp.float32),
    mesh=_MESH,
    scratch_types=[
        pltpu.VMEM_SHARED((NPAD,), jnp.float32),
        pltpu.VMEM((256, KD), jnp.int32),
        pltpu.VMEM((KD,), jnp.float32),
        pltpu.VMEM((NPAD // NS,), jnp.float32),
        pltpu.VMEM((NPAD // NW,), jnp.float32),
    ],
)
def _deg_dis(dst2_hbm, dis_hbm, deg_sp, dv_all, ones_v, zv, wv):
    c = lax.axis_index("c")
    s = lax.axis_index("s")
    for j in range(NPAD // NS // 16):
        zv[pl.ds(j * 16, 16)] = jnp.zeros((16,), jnp.float32)
    for j in range(KD // 16):
        ones_v[pl.ds(j * 16, 16)] = jnp.ones((16,), jnp.float32)
    pltpu.sync_copy(zv, deg_sp.at[pl.ds(s * (NPAD // NS), NPAD // NS)])
    plsc.subcore_barrier()

    cnt = jnp.where(s < 12, 248, 256)
    base = jnp.where(s < 12, s * 248, 2976 + (s - 12) * 256)
    lbase = jnp.minimum(base, NCHUNK - 256)
    off = base - lbase
    pltpu.sync_copy(dst2_hbm.at[pl.ds(lbase, 256)], dv_all)

    def body(j, carry):
        pltpu.sync_copy(ones_v, deg_sp.at[dv_all.at[off + j]], add=True)
        return carry

    lax.fori_loop(0, cnt, body, 0)
    plsc.subcore_barrier()

    w = c * NS + s
    nb = NPAD // NW
    pltpu.sync_copy(deg_sp.at[pl.ds(w * nb, nb)], wv)
    for g in range(nb // 16):
        d = wv[pl.ds(g * 16, 16)] + 1.0
        wv[pl.ds(g * 16, 16)] = _rsqrt16(d)
    pltpu.sync_copy(wv, dis_hbm.at[pl.ds(w * nb, nb)])


def _make_agg(F, K, nlo, clo, chi, tc_tiling=True):
    nchunk_tot = E // K
    assert nlo * clo + (NW - nlo) * chi == nchunk_tot

    @functools.partial(
        pl.kernel,
        out_type=jax.ShapeDtypeStruct((NC, NPAD, F), jnp.float32),
        mesh=_MESH,
        compiler_params=pltpu.CompilerParams(use_tc_tiling_on_sc=tc_tiling),
        scratch_types=[
            pltpu.VMEM_SHARED((NPAD, F), jnp.float32),
            pltpu.VMEM((chi * K,), jnp.int32),
            pltpu.VMEM((chi, K), jnp.int32),
            pltpu.VMEM((K, F), jnp.float32),
            pltpu.VMEM((K, F), jnp.float32),
            pltpu.SemaphoreType.DMA,
            pltpu.SemaphoreType.DMA,
        ],
    )
    def _agg(h_hbm, src_hbm, dst2_hbm, zero_hbm, out_hbm, acc_sp,
             sv_all, dv_all, rows0, rows1, gs0, gs1):
        c = lax.axis_index("c")
        s = lax.axis_index("s")
        pltpu.sync_copy(zero_hbm, acc_sp.at[pl.ds(s * ROWS_PER_TILE, ROWS_PER_TILE)])
        w = c * NS + s
        cnt = jnp.where(w < nlo, clo, chi)
        base = jnp.where(w < nlo, w * clo, nlo * clo + (w - nlo) * chi)
        lbase = jnp.minimum(base, nchunk_tot - chi)
        off = base - lbase
        pltpu.sync_copy(src_hbm.at[pl.ds(lbase * K, chi * K)], sv_all)
        pltpu.sync_copy(dst2_hbm.at[pl.ds(lbase, chi)], dv_all)
        plsc.subcore_barrier()

        def gather(rel, rows, sem):
            return pltpu.async_copy(
                h_hbm.at[sv_all.at[pl.ds((off + rel) * K, K)]], rows, sem
            )

        gather(0, rows0, gs0).wait()

        def body(p, carry):
            r0 = 2 * p
            d_g1 = gather(r0 + 1, rows1, gs1)
            pltpu.sync_copy(rows0, acc_sp.at[dv_all.at[off + r0]], add=True)
            d_g1.wait()
            r2 = jnp.minimum(r0 + 2, cnt - 2)
            d_g0 = gather(r2, rows0, gs0)
            pltpu.sync_copy(rows1, acc_sp.at[dv_all.at[off + r0 + 1]], add=True)
            d_g0.wait()
            return carry

        lax.fori_loop(0, cnt // 2, body, 0)
        plsc.subcore_barrier()
        pltpu.sync_copy(
            acc_sp.at[pl.ds(s * ROWS_PER_TILE, ROWS_PER_TILE)],
            out_hbm.at[c, pl.ds(s * ROWS_PER_TILE, ROWS_PER_TILE)],
        )

    return _agg


_agg128 = _make_agg(FH, 80, 12, 120, 128)
_agg32 = _make_agg(OP, 128, 30, 78, 80, tc_tiling=False)


RB = 1000
GRID = N // RB


def _tc0_body(x_ref, w_ref, o_ref):
    o_ref[...] = jnp.dot(x_ref[...], w_ref[...], preferred_element_type=jnp.float32)


def _tc1_body(m_ref, d_ref, o_ref):
    o_ref[...] = m_ref[...] * d_ref[...]


def _tc2_body(p_ref, h_ref, d_ref, b_ref, w_ref, o_ref):
    d = d_ref[...]
    y = jnp.maximum((p_ref[0] + p_ref[1] + h_ref[...]) * d + b_ref[...], 0.0)
    o_ref[...] = jnp.dot(y, w_ref[...], preferred_element_type=jnp.float32) * d


def _tc3_body(p_ref, h_ref, d_ref, b_ref, o_ref):
    o_ref[...] = (p_ref[0] + p_ref[1] + h_ref[...]) * d_ref[...] + b_ref[...]


def _row_spec(F):
    return pl.BlockSpec((RB, F), lambda i: (i, 0))


def _full_spec(shape):
    n = len(shape)
    return pl.BlockSpec(shape, lambda i, _n=n: (0,) * _n)


def _parts_spec(F):
    return pl.BlockSpec((NC, RB, F), lambda i: (0, i, 0))


def _tc0(x, W1):
    return pl.pallas_call(
        _tc0_body,
        grid=(GRID,),
        in_specs=[_row_spec(D), _full_spec((D, FH))],
        out_specs=_row_spec(FH),
        out_shape=jax.ShapeDtypeStruct((N, FH), jnp.float32),
    )(x, W1)


def _tc1(m1, dis):
    return pl.pallas_call(
        _tc1_body,
        grid=(GRID,),
        in_specs=[_row_spec(FH), _row_spec(1)],
        out_specs=_row_spec(FH),
        out_shape=jax.ShapeDtypeStruct((N, FH), jnp.float32),
    )(m1, dis)


def _tc2(p1, h1p, dis, b1r, W2p):
    return pl.pallas_call(
        _tc2_body,
        grid=(GRID,),
        in_specs=[
            _parts_spec(FH),
            _row_spec(FH),
            _row_spec(1),
            _full_spec((1, FH)),
            _full_spec((FH, OP)),
        ],
        out_specs=_row_spec(OP),
        out_shape=jax.ShapeDtypeStruct((N, OP), jnp.float32),
    )(p1, h1p, dis, b1r, W2p)


def _tc3(p2, h2p, dis, b2r):
    return pl.pallas_call(
        _tc3_body,
        grid=(GRID,),
        in_specs=[
            _parts_spec(OP),
            _row_spec(OP),
            _row_spec(1),
            _full_spec((1, OP)),
        ],
        out_specs=_row_spec(OP),
        out_shape=jax.ShapeDtypeStruct((N, OP), jnp.float32),
    )(p2, h2p, dis, b2r)


def kernel(x, edge_index, W1, b1, W2, b2):
    x = x.astype(jnp.float32)
    src = edge_index[0].astype(jnp.int32)
    dst = edge_index[1].astype(jnp.int32)

    dst2 = dst.reshape(NCHUNK, KD)
    dis_pad = _deg_dis(dst2)
    dis = dis_pad[:N].reshape(N, 1)
    m1 = _tc0(x, W1)
    h1p = _tc1(m1, dis)

    z128 = jnp.zeros((ROWS_PER_TILE, FH), jnp.float32)
    p1 = _agg128(h1p, src, dst2, z128)

    b1r = b1.reshape(1, FH)
    W2p = jnp.zeros((FH, OP), jnp.float32).at[:, :OUTD].set(W2)
    h2p = _tc2(p1, h1p, dis, b1r, W2p)

    z32 = jnp.zeros((ROWS_PER_TILE, OP), jnp.float32)
    dst3 = dst.reshape(E // 128, 128)
    p2 = _agg32(h2p, src, dst3, z32)

    b2r = jnp.zeros((1, OP), jnp.float32).at[0, :OUTD].set(b2)
    outp = _tc3(p2, h2p, dis, b2r)
    return outp[:, :OUTD]


# speedup vs baseline: 29.1068x; 1.0216x over previous
import functools

import jax
import jax.numpy as jnp
from jax import lax
from jax.experimental import pallas as pl
from jax.experimental.pallas import tpu as pltpu
from jax.experimental.pallas import tpu_sc as plsc

N = 10000
E = 320000
D = 128
FH = 128
OUTD = 21
OP = 32
NC = 2
NS = 16
NW = NC * NS
NPAD = 10240
ROWS_PER_TILE = NPAD // NS
EPW = E // NW

_MESH = plsc.VectorSubcoreMesh(
    core_axis_name="c", subcore_axis_name="s", num_cores=NC, num_subcores=NS
)


def _rsqrt16(x):
    y = 1.0 / x
    hx = 0.5 * x
    for _ in range(26):
        y = y * (1.5 - hx * y * y)
    return y


KD = 128
NCHUNK = E // KD


@functools.partial(
    pl.kernel,
    out_type=jax.ShapeDtypeStruct((NPAD,), jnp.float32),
    mesh=_MESH,
    compiler_params=pltpu.CompilerParams(use_tc_tiling_on_sc=False),
    scratch_types=[
        pltpu.VMEM_SHARED((NPAD,), jnp.float32),
        pltpu.VMEM((157, KD), jnp.int32),
        pltpu.VMEM((KD,), jnp.float32),
        pltpu.VMEM((NPAD // NS,), jnp.float32),
        pltpu.VMEM((NPAD // NW,), jnp.float32),
    ],
)
def _deg_dis(dst2_hbm, dis_hbm, deg_sp, dv_all, ones_v, zv, wv):
    c = lax.axis_index("c")
    s = lax.axis_index("s")
    for j in range(NPAD // NS // 16):
        zv[pl.ds(j * 16, 16)] = jnp.zeros((16,), jnp.float32)
    for j in range(KD // 16):
        ones_v[pl.ds(j * 16, 16)] = jnp.ones((16,), jnp.float32)
    pltpu.sync_copy(zv, deg_sp.at[pl.ds(s * (NPAD // NS), NPAD // NS)])
    plsc.subcore_barrier()

    cnt = jnp.where(s < 12, 156, 157)
    base = jnp.where(s < 12, s * 156, 1872 + (s - 12) * 157)
    lbase = jnp.minimum(base, NCHUNK - 157)
    off = base - lbase
    pltpu.sync_copy(dst2_hbm.at[pl.ds(lbase, 157)], dv_all)

    def body(j, carry):
        pltpu.sync_copy(ones_v, deg_sp.at[dv_all.at[off + j]], add=True)
        return carry

    lax.fori_loop(0, cnt, body, 0)
    plsc.subcore_barrier()

    w = c * NS + s
    nb = NPAD // NW
    pltpu.sync_copy(deg_sp.at[pl.ds(w * nb, nb)], wv)
    for g in range(nb // 16):
        d = wv[pl.ds(g * 16, 16)] + 1.0
        wv[pl.ds(g * 16, 16)] = _rsqrt16(d)
    pltpu.sync_copy(wv, dis_hbm.at[pl.ds(w * nb, nb)])


def _make_agg(F, K, nlo, clo, chi, tc_tiling=True):
    nchunk_tot = E // K
    assert nlo * clo + (NW - nlo) * chi == nchunk_tot

    @functools.partial(
        pl.kernel,
        out_type=jax.ShapeDtypeStruct((NC, NPAD, F), jnp.float32),
        mesh=_MESH,
        compiler_params=pltpu.CompilerParams(use_tc_tiling_on_sc=tc_tiling),
        scratch_types=[
            pltpu.VMEM_SHARED((NPAD, F), jnp.float32),
            pltpu.VMEM((chi * K,), jnp.int32),
            pltpu.VMEM((chi, K), jnp.int32),
            pltpu.VMEM((K, F), jnp.float32),
            pltpu.VMEM((K, F), jnp.float32),
            pltpu.SemaphoreType.DMA,
            pltpu.SemaphoreType.DMA,
        ],
    )
    def _agg(h_hbm, src_hbm, dst2_hbm, zero_hbm, out_hbm, acc_sp,
             sv_all, dv_all, rows0, rows1, gs0, gs1):
        c = lax.axis_index("c")
        s = lax.axis_index("s")
        pltpu.sync_copy(zero_hbm, acc_sp.at[pl.ds(s * ROWS_PER_TILE, ROWS_PER_TILE)])
        w = c * NS + s
        cnt = jnp.where(w < nlo, clo, chi)
        base = jnp.where(w < nlo, w * clo, nlo * clo + (w - nlo) * chi)
        lbase = jnp.minimum(base, nchunk_tot - chi)
        off = base - lbase
        pltpu.sync_copy(src_hbm.at[pl.ds(lbase * K, chi * K)], sv_all)
        pltpu.sync_copy(dst2_hbm.at[pl.ds(lbase, chi)], dv_all)
        plsc.subcore_barrier()

        def gather(rel, rows, sem):
            return pltpu.async_copy(
                h_hbm.at[sv_all.at[pl.ds((off + rel) * K, K)]], rows, sem
            )

        gather(0, rows0, gs0).wait()

        def body(p, carry):
            r0 = 2 * p
            d_g1 = gather(r0 + 1, rows1, gs1)
            pltpu.sync_copy(rows0, acc_sp.at[dv_all.at[off + r0]], add=True)
            d_g1.wait()
            r2 = jnp.minimum(r0 + 2, cnt - 2)
            d_g0 = gather(r2, rows0, gs0)
            pltpu.sync_copy(rows1, acc_sp.at[dv_all.at[off + r0 + 1]], add=True)
            d_g0.wait()
            return carry

        lax.fori_loop(0, cnt // 2, body, 0)
        plsc.subcore_barrier()
        pltpu.sync_copy(
            acc_sp.at[pl.ds(s * ROWS_PER_TILE, ROWS_PER_TILE)],
            out_hbm.at[c, pl.ds(s * ROWS_PER_TILE, ROWS_PER_TILE)],
        )

    return _agg


_agg128 = _make_agg(FH, 80, 12, 120, 128)
_agg32 = _make_agg(OP, 128, 30, 78, 80, tc_tiling=False)


RB = 1000
GRID = N // RB


def _tc1_body(x_ref, w_ref, d_ref, o_ref):
    m = jnp.dot(x_ref[...], w_ref[...], preferred_element_type=jnp.float32)
    o_ref[...] = m * d_ref[...]


def _tc2_body(p_ref, h_ref, d_ref, b_ref, w_ref, o_ref):
    d = d_ref[...]
    y = jnp.maximum((p_ref[0] + p_ref[1] + h_ref[...]) * d + b_ref[...], 0.0)
    o_ref[...] = jnp.dot(y, w_ref[...], preferred_element_type=jnp.float32) * d


def _tc3_body(p_ref, h_ref, d_ref, b_ref, o_ref):
    full = (p_ref[0] + p_ref[1] + h_ref[...]) * d_ref[...] + b_ref[...]
    o_ref[...] = full[:, :OUTD]


def _row_spec(F):
    return pl.BlockSpec((RB, F), lambda i: (i, 0))


def _full_spec(shape):
    n = len(shape)
    return pl.BlockSpec(shape, lambda i, _n=n: (0,) * _n)


def _parts_spec(F):
    return pl.BlockSpec((NC, RB, F), lambda i: (0, i, 0))


def _tc1(x, W1, dis):
    return pl.pallas_call(
        _tc1_body,
        grid=(GRID,),
        in_specs=[_row_spec(D), _full_spec((D, FH)), _row_spec(1)],
        out_specs=_row_spec(FH),
        out_shape=jax.ShapeDtypeStruct((N, FH), jnp.float32),
    )(x, W1, dis)


def _tc2(p1, h1p, dis, b1r, W2p):
    return pl.pallas_call(
        _tc2_body,
        grid=(GRID,),
        in_specs=[
            _parts_spec(FH),
            _row_spec(FH),
            _row_spec(1),
            _full_spec((1, FH)),
            _full_spec((FH, OP)),
        ],
        out_specs=_row_spec(OP),
        out_shape=jax.ShapeDtypeStruct((N, OP), jnp.float32),
    )(p1, h1p, dis, b1r, W2p)


def _tc3(p2, h2p, dis, b2r):
    return pl.pallas_call(
        _tc3_body,
        grid=(GRID,),
        in_specs=[
            _parts_spec(OP),
            _row_spec(OP),
            _row_spec(1),
            _full_spec((1, OP)),
        ],
        out_specs=_row_spec(OUTD),
        out_shape=jax.ShapeDtypeStruct((N, OUTD), jnp.float32),
    )(p2, h2p, dis, b2r)


def kernel(x, edge_index, W1, b1, W2, b2):
    x = x.astype(jnp.float32)
    src = edge_index[0].astype(jnp.int32)
    dst = edge_index[1].astype(jnp.int32)

    dst2 = dst.reshape(E // 80, 80)
    dst3 = dst.reshape(NCHUNK, KD)
    dis_pad = _deg_dis(dst3)
    dis = dis_pad[:N].reshape(N, 1)
    h1p = _tc1(x, W1, dis)

    z128 = jnp.zeros((ROWS_PER_TILE, FH), jnp.float32)
    p1 = _agg128(h1p, src, dst2, z128)

    b1r = b1.reshape(1, FH)
    W2p = jnp.zeros((FH, OP), jnp.float32).at[:, :OUTD].set(W2)
    h2p = _tc2(p1, h1p, dis, b1r, W2p)

    z32 = jnp.zeros((ROWS_PER_TILE, OP), jnp.float32)
    p2 = _agg32(h2p, src, dst3, z32)

    b2r = jnp.zeros((1, OP), jnp.float32).at[0, :OUTD].set(b2)
    return _tc3(p2, h2p, dis, b2r)
